# baseline (speedup 1.0000x reference)
import functools

import jax
import jax.numpy as jnp
from jax.experimental import pallas as pl
from jax.experimental.pallas import tpu as pltpu

_EPS = 1e-5


def _ln_kernel(x_ref, pe_ref, gamma_ref, beta_ref, out_ref):
    h = x_ref[...] + pe_ref[...][None, :, :]
    mean = jnp.mean(h, axis=-1, keepdims=True)
    cent = h - mean
    var = jnp.mean(cent * cent, axis=-1, keepdims=True)
    inv = jax.lax.rsqrt(var + _EPS)
    out_ref[...] = cent * inv * gamma_ref[...][None, None, :] + beta_ref[...][None, None, :]


@functools.partial(jax.jit, static_argnames=("blk",))
def _fused_embed_ln(x, pos_embed_weight, ln_gamma, ln_beta, blk):
    batch, nb_seq, d_em = x.shape
    grid = (nb_seq // blk,)
    return pl.pallas_call(
        _ln_kernel,
        grid=grid,
        in_specs=[
            pl.BlockSpec((batch, blk, d_em), lambda j: (0, j, 0)),
            pl.BlockSpec((blk, d_em), lambda j: (j, 0)),
            pl.BlockSpec((d_em,), lambda j: (0,)),
            pl.BlockSpec((d_em,), lambda j: (0,)),
        ],
        out_specs=pl.BlockSpec((batch, blk, d_em), lambda j: (0, j, 0)),
        out_shape=jax.ShapeDtypeStruct((batch, nb_seq, d_em), x.dtype),
        compiler_params=pltpu.CompilerParams(
            dimension_semantics=("parallel",),
        ),
    )(x, pos_embed_weight, ln_gamma, ln_beta)


def kernel(x, pos_embed_weight, ln_gamma, ln_beta, batch_size):
    del batch_size
    return _fused_embed_ln(x, pos_embed_weight, ln_gamma, ln_beta, blk=256)


# speedup vs baseline: 1.0346x; 1.0346x over previous
import functools

import jax
import jax.numpy as jnp
from jax.experimental import pallas as pl
from jax.experimental.pallas import tpu as pltpu

_EPS = 1e-5


def _ln_kernel(x_ref, pe_ref, gamma_ref, beta_ref, out_ref):
    h = x_ref[...] + pe_ref[...][None, :, :]
    mean = jnp.mean(h, axis=-1, keepdims=True)
    cent = h - mean
    var = jnp.mean(cent * cent, axis=-1, keepdims=True)
    inv = jax.lax.rsqrt(var + _EPS)
    out_ref[...] = cent * inv * gamma_ref[...][None, None, :] + beta_ref[...][None, None, :]


@functools.partial(jax.jit, static_argnames=("blk",))
def _fused_embed_ln(x, pos_embed_weight, ln_gamma, ln_beta, blk):
    batch, nb_seq, d_em = x.shape
    grid = (nb_seq // blk,)
    return pl.pallas_call(
        _ln_kernel,
        grid=grid,
        in_specs=[
            pl.BlockSpec((batch, blk, d_em), lambda j: (0, j, 0)),
            pl.BlockSpec((blk, d_em), lambda j: (j, 0)),
            pl.BlockSpec((d_em,), lambda j: (0,)),
            pl.BlockSpec((d_em,), lambda j: (0,)),
        ],
        out_specs=pl.BlockSpec((batch, blk, d_em), lambda j: (0, j, 0)),
        out_shape=jax.ShapeDtypeStruct((batch, nb_seq, d_em), x.dtype),
        compiler_params=pltpu.CompilerParams(
            dimension_semantics=("parallel",),
        ),
    )(x, pos_embed_weight, ln_gamma, ln_beta)


def kernel(x, pos_embed_weight, ln_gamma, ln_beta, batch_size):
    del batch_size
    return _fused_embed_ln(x, pos_embed_weight, ln_gamma, ln_beta, blk=512)
